# baseline (speedup 1.0000x reference)
import functools

import jax
import jax.numpy as jnp
from jax import lax
from jax.experimental import pallas as pl
from jax.experimental.pallas import tpu as pltpu
from jax.experimental.pallas import tpu_sc as plsc

F32 = jnp.float32
I32 = jnp.int32

N_NODE = 10000
E_T = 160000
N_TYPES = 3
E_ALL = N_TYPES * E_T
H = 128
ACT = 8

NC = 2
NS = 16
NW = NC * NS
LANES = 16

RNG = 320
NPAD = NW * RNG
ACCROWS = RNG + 1
SCRAP = RNG
SCAN_CHUNK = 2000
N_SCAN = E_T // SCAN_CHUNK
FLUSH = 128
CAP_CH = 2048
CAP = N_SCAN * CAP_CH
RMW_CHUNK = 128
EPW = E_ALL // NW
GCH = 120
BLK_E = 1600
BLK_N = 2000
INTERPRET = False


def _wid():
    return lax.axis_index("s") * NC + lax.axis_index("c")


def _mesh():
    return plsc.VectorSubcoreMesh(core_axis_name="c", subcore_axis_name="s",
                                  num_cores=NC, num_subcores=NS)


_SC_PARAMS = pltpu.CompilerParams(needs_layout_passes=False)


def _match_body(dstall, mlist, mcnt, dstbuf, mbuf, cbuf):
    w = _wid()
    iota = lax.iota(I32, 16)
    node_base = w * RNG
    sentinel = jnp.zeros((16,), I32) + SCRAP

    def per_type(t, _):
        def per_chunk(c, G):
            pltpu.sync_copy(
                dstall.at[pl.ds(pl.multiple_of(t * E_T + c * SCAN_CHUNK, 8),
                                SCAN_CHUNK)],
                dstbuf)

            def scan(i, cntv):
                idx = i * 16 + iota
                v = plsc.load_gather(dstbuf, [idx])
                dl = v - node_base
                mask = (dl >= 0) & (dl < RNG)
                eidv = t * E_T + c * SCAN_CHUNK + idx
                packed = (eidv << 9) + dl
                pos = cntv + plsc.cumsum(mask.astype(I32)) - 1
                plsc.store_scatter(mbuf, [pos], packed, mask=mask)
                return cntv + plsc.all_reduce_population_count(mask)

            cntv = lax.fori_loop(0, SCAN_CHUNK // 16, scan,
                                 jnp.zeros((16,), I32))
            cnt = jnp.max(cntv)
            for k in range(FLUSH // 16):
                plsc.store_scatter(mbuf, [cnt + k * 16 + iota], sentinel)
            nflush = (cnt + (FLUSH - 1)) // FLUSH

            def flush(k, _):
                pltpu.sync_copy(
                    mbuf.at[pl.ds(k * FLUSH, FLUSH)],
                    mlist.at[pl.ds(
                        pl.multiple_of(
                            (t * NW + w) * CAP + G + k * FLUSH, 8),
                        FLUSH)])
                return 0

            lax.fori_loop(0, nflush, flush, 0)
            return G + nflush * FLUSH

        G = lax.fori_loop(0, N_SCAN, per_chunk, 0)
        cbuf[...] = jnp.zeros((16,), I32) + G
        pltpu.sync_copy(
            cbuf, mcnt.at[pl.ds(pl.multiple_of((t * NW + w) * 16, 8), 16)])
        return 0

    lax.fori_loop(0, N_TYPES, per_type, 0)


def _match_call(dstall):
    return pl.kernel(
        _match_body,
        compiler_params=_SC_PARAMS,
        interpret=INTERPRET,
        out_type=(jax.ShapeDtypeStruct((N_TYPES * NW * CAP,), I32),
                  jax.ShapeDtypeStruct((N_TYPES * NW * 16,), I32)),
        mesh=_mesh(),
        scratch_types=[
            pltpu.VMEM((SCAN_CHUNK,), I32),
            pltpu.VMEM((SCAN_CHUNK + FLUSH,), I32),
            pltpu.VMEM((16,), I32),
        ],
    )(dstall)


def _seg_body(msg, mlist, mcnt, agg, acc, outacc, pbuf, idxbuf, dstlbuf,
              rows, cntbuf, sem):
    w = _wid()
    iota = lax.iota(I32, 16)
    neg = jnp.full((16,), -jnp.inf, F32)

    def init_out(i, _):
        plsc.store_scatter(outacc, [i * 16 + iota], neg)
        return 0

    lax.fori_loop(0, (RNG * H) // 16, init_out, 0)

    def per_type(t, _):
        def init_acc(i, _):
            plsc.store_scatter(acc, [i * 16 + iota], neg)
            return 0

        lax.fori_loop(0, (ACCROWS * H) // 16, init_acc, 0)
        pltpu.sync_copy(
            mcnt.at[pl.ds(pl.multiple_of((t * NW + w) * 16, 8), 16)], cntbuf)
        cnt = jnp.max(cntbuf[...])

        def per_chunk(c, _):
            pltpu.sync_copy(
                mlist.at[pl.ds(
                    pl.multiple_of((t * NW + w) * CAP + c * RMW_CHUNK, 8),
                    RMW_CHUNK)],
                pbuf)
            for i in range(RMW_CHUNK // 16):
                v = pbuf[pl.ds(i * 16, 16)]
                idxbuf[pl.ds(i * 16, 16)] = v >> 9
                dstlbuf[pl.ds(i * 16, 16)] = v & 511
            pltpu.async_copy(msg.at[idxbuf], rows, sem).wait()

            def rmw(j, _):
                js = jnp.zeros((16,), I32) + j
                dstv = plsc.load_gather(dstlbuf, [js])
                base = dstv * H
                for f in range(H // 16):
                    col = f * 16 + iota
                    r = plsc.load_gather(rows, [js, col])
                    addr = base + col
                    a = plsc.load_gather(acc, [addr])
                    plsc.store_scatter(acc, [addr], jnp.maximum(a, r))
                return 0

            lax.fori_loop(0, RMW_CHUNK, rmw, 0)
            return 0

        lax.fori_loop(0, cnt // RMW_CHUNK, per_chunk, 0)

        def merge(i, _):
            ad = i * 16 + iota
            a = plsc.load_gather(acc, [ad])
            o = plsc.load_gather(outacc, [ad])
            a0 = jnp.where(a == -jnp.inf, jnp.zeros((16,), F32), a)
            plsc.store_scatter(outacc, [ad], jnp.maximum(o, a0))
            return 0

        lax.fori_loop(0, (RNG * H) // 16, merge, 0)
        return 0

    lax.fori_loop(0, N_TYPES, per_type, 0)
    pltpu.sync_copy(
        outacc, agg.at[pl.ds(pl.multiple_of(w * RNG * H, 8), RNG * H)])


def _seg_call(msg, mlist, mcnt):
    return pl.kernel(
        _seg_body,
        compiler_params=_SC_PARAMS,
        interpret=INTERPRET,
        out_type=jax.ShapeDtypeStruct((NPAD * H,), F32),
        mesh=_mesh(),
        scratch_types=[
            pltpu.VMEM((ACCROWS * H,), F32),
            pltpu.VMEM((RNG * H,), F32),
            pltpu.VMEM((RMW_CHUNK,), I32),
            pltpu.VMEM((RMW_CHUNK,), I32),
            pltpu.VMEM((RMW_CHUNK,), I32),
            pltpu.VMEM((RMW_CHUNK, H), F32),
            pltpu.VMEM((16,), I32),
            pltpu.SemaphoreType.DMA,
        ],
    )(msg, mlist, mcnt)


def _gather_body(ya, dstall, g, idxb, rows, sem):
    w = _wid()

    def per_chunk(c, _):
        base = pl.multiple_of(w * EPW + c * GCH, 8)
        pltpu.sync_copy(dstall.at[pl.ds(base, GCH)], idxb)
        pltpu.async_copy(ya.at[idxb], rows, sem).wait()
        pltpu.sync_copy(rows, g.at[pl.ds(base, GCH)])
        return 0

    lax.fori_loop(0, EPW // GCH, per_chunk, 0)


def _gather_call(ya, dstall):
    return pl.kernel(
        _gather_body,
        compiler_params=_SC_PARAMS,
        interpret=INTERPRET,
        out_type=jax.ShapeDtypeStruct((E_ALL, H), F32),
        mesh=_mesh(),
        scratch_types=[
            pltpu.VMEM((GCH,), I32),
            pltpu.VMEM((GCH, H), F32),
            pltpu.SemaphoreType.DMA,
        ],
    )(ya, dstall)


def _dot(a, b):
    return jnp.dot(a, b, preferred_element_type=F32)


def _full(shape):
    return pl.BlockSpec(shape, lambda i: (0, 0))


def _embed_kernel(x, w0, b0, w1, b1, o):
    h = jnp.maximum(_dot(x[...], w0[...]) + b0[...], 0.0)
    o[...] = _dot(h, w1[...]) + b1[...]


def _embed_call(x, w0, b0, w1, b1, blk):
    n, d = x.shape
    return pl.pallas_call(
        _embed_kernel,
        grid=(n // blk,),
        in_specs=[pl.BlockSpec((blk, d), lambda i: (i, 0)),
                  _full(w0.shape), _full(b0.shape),
                  _full(w1.shape), _full(b1.shape)],
        out_specs=pl.BlockSpec((blk, H), lambda i: (i, 0)),
        out_shape=jax.ShapeDtypeStruct((n, H), F32),
        interpret=INTERPRET,
    )(x, w0, b0, w1, b1)


def _edge0_kernel(x, w0, b0, w1, b1, fw0, fb0, fw1, fb1, ea, msg):
    h = jnp.maximum(_dot(x[...], w0[...]) + b0[...], 0.0)
    e = _dot(h, w1[...]) + b1[...]
    ea[...] = e
    h2 = jnp.maximum(_dot(e, fw0[...]) + fb0[...], 0.0)
    msg[...] = _dot(h2, fw1[...]) + fb1[...]


def _edge0_call(x, w0, b0, w1, b1, fw0, fb0, fw1, fb1, blk):
    n, d = x.shape
    return pl.pallas_call(
        _edge0_kernel,
        grid=(n // blk,),
        in_specs=[pl.BlockSpec((blk, d), lambda i: (i, 0)),
                  _full(w0.shape), _full(b0.shape),
                  _full(w1.shape), _full(b1.shape),
                  _full(fw0.shape), _full(fb0.shape),
                  _full(fw1.shape), _full(fb1.shape)],
        out_specs=[pl.BlockSpec((blk, H), lambda i: (i, 0)),
                   pl.BlockSpec((blk, H), lambda i: (i, 0))],
        out_shape=[jax.ShapeDtypeStruct((n, H), F32),
                   jax.ShapeDtypeStruct((n, H), F32)],
        interpret=INTERPRET,
    )(x, w0, b0, w1, b1, fw0, fb0, fw1, fb1)


def _node_kernel(xa, agg, w0b, b0, xan, ya):
    x = xa[...] + agg[...]
    xan[...] = x
    ya[...] = _dot(x, w0b[...]) + b0[...]


def _node_call(xa, agg, w0b, b0, blk):
    n = xa.shape[0]
    return pl.pallas_call(
        _node_kernel,
        grid=(n // blk,),
        in_specs=[pl.BlockSpec((blk, H), lambda i: (i, 0)),
                  pl.BlockSpec((blk, H), lambda i: (i, 0)),
                  _full(w0b.shape), _full(b0.shape)],
        out_specs=[pl.BlockSpec((blk, H), lambda i: (i, 0)),
                   pl.BlockSpec((blk, H), lambda i: (i, 0))],
        out_shape=[jax.ShapeDtypeStruct((n, H), F32),
                   jax.ShapeDtypeStruct((n, H), F32)],
        interpret=INTERPRET,
    )(xa, agg, w0b, b0)


def _edge_kernel(ea, g, w0a, w1, b1, fw0, fb0, fw1, fb1, ean, msg):
    e = ea[...]
    t = jnp.maximum(_dot(e, w0a[...]) + g[...], 0.0)
    e = e + _dot(t, w1[...]) + b1[...]
    ean[...] = e
    h = jnp.maximum(_dot(e, fw0[...]) + fb0[...], 0.0)
    msg[...] = _dot(h, fw1[...]) + fb1[...]


def _edge_call(ea, g, w0a, w1, b1, fw0, fb0, fw1, fb1, blk):
    n = ea.shape[0]
    return pl.pallas_call(
        _edge_kernel,
        grid=(n // blk,),
        in_specs=[pl.BlockSpec((blk, H), lambda i: (i, 0)),
                  pl.BlockSpec((blk, H), lambda i: (i, 0)),
                  _full(w0a.shape), _full(w1.shape), _full(b1.shape),
                  _full(fw0.shape), _full(fb0.shape),
                  _full(fw1.shape), _full(fb1.shape)],
        out_specs=[pl.BlockSpec((blk, H), lambda i: (i, 0)),
                   pl.BlockSpec((blk, H), lambda i: (i, 0))],
        out_shape=[jax.ShapeDtypeStruct((n, H), F32),
                   jax.ShapeDtypeStruct((n, H), F32)],
        interpret=INTERPRET,
    )(ea, g, w0a, w1, b1, fw0, fb0, fw1, fb1)


def _field_kernel(xa, agg, act, w0x, w0a, b0, w1p, b1p, o):
    x = xa[...] + agg[...]
    h = jnp.maximum(_dot(x, w0x[...]) + _dot(act[...], w0a[...]) + b0[...],
                    0.0)
    o[...] = _dot(h, w1p[...]) + b1p[...]


def _field_call(xa, agg, act, w0x, w0a, b0, w1p, b1p, blk):
    n = xa.shape[0]
    return pl.pallas_call(
        _field_kernel,
        grid=(n // blk,),
        in_specs=[pl.BlockSpec((blk, H), lambda i: (i, 0)),
                  pl.BlockSpec((blk, H), lambda i: (i, 0)),
                  pl.BlockSpec((blk, ACT), lambda i: (i, 0)),
                  _full(w0x.shape), _full(w0a.shape), _full(b0.shape),
                  _full(w1p.shape), _full(b1p.shape)],
        out_specs=pl.BlockSpec((blk, H), lambda i: (i, 0)),
        out_shape=jax.ShapeDtypeStruct((n, H), F32),
        interpret=INTERPRET,
    )(xa, agg, act, w0x, w0a, b0, w1p, b1p)


def kernel(x_obstacle, x_agent, x_goal, edge_index_oa, edge_index_aa,
           edge_index_ga, edge_attr_oa, edge_attr_aa, edge_attr_ga, action,
           params):
    p = params
    r1 = lambda b: b.reshape(1, -1)

    dst3 = jnp.stack([edge_index_oa[1], edge_index_aa[1], edge_index_ga[1]])
    dst_all = dst3.reshape(-1)
    ecat = jnp.concatenate([edge_attr_oa, edge_attr_aa, edge_attr_ga], axis=0)

    xa = _embed_call(x_agent, p["embed_W0"], r1(p["embed_b0"]),
                     p["embed_W1"], r1(p["embed_b1"]), BLK_N)

    ea, msg = _edge0_call(ecat, p["eembed_W0"], r1(p["eembed_b0"]),
                          p["eembed_W1"], r1(p["eembed_b1"]),
                          p["fx0_W0"], r1(p["fx0_b0"]),
                          p["fx0_W1"], r1(p["fx0_b1"]), BLK_E)

    mlist, mcnt = _match_call(dst_all)

    for l in range(3):
        aggf = _seg_call(msg, mlist, mcnt)
        agg = aggf.reshape(NPAD, H)[:N_NODE]
        if l == 2:
            w0 = p["field_W0"]
            w1p = jnp.pad(p["field_W1"], ((0, 0), (0, H - 1)))
            b1p = jnp.broadcast_to(p["field_b1"].reshape(1, 1), (1, H))
            out = _field_call(xa, agg, action, w0[:H], w0[H:],
                              r1(p["field_b0"]), w1p, b1p, BLK_N)
            return out[:, 0]
        em = "em%d_" % l
        w0 = p[em + "W0"]
        xa, ya = _node_call(xa, agg, w0[H:], r1(p[em + "b0"]), BLK_N)
        g = _gather_call(ya, dst_all)
        fx = "fx%d_" % (l + 1)
        ea, msg = _edge_call(ea, g, w0[:H], p[em + "W1"], r1(p[em + "b1"]),
                             p[fx + "W0"], r1(p[fx + "b0"]),
                             p[fx + "W1"], r1(p[fx + "b1"]), BLK_E)


# speedup vs baseline: 14.4194x; 14.4194x over previous
import functools

import jax
import jax.numpy as jnp
from jax import lax
from jax.experimental import pallas as pl
from jax.experimental.pallas import tpu as pltpu
from jax.experimental.pallas import tpu_sc as plsc

F32 = jnp.float32
I32 = jnp.int32

N_NODE = 10000
E_T = 160000
N_TYPES = 3
E_ALL = N_TYPES * E_T
H = 128
ACT = 8

NC = 2
NS = 16
NW = NC * NS

RNG = 320
NPAD = NW * RNG
ACCROWS = RNG + 1
SCRAP = RNG
SCAN_CHUNK = 2000
N_SCAN = E_T // SCAN_CHUNK
ROUND = 16384
MB_CAP = ROUND + SCAN_CHUNK + 160
HB = 336
CAP = 160384
RMW_CHUNK = 128
EPW = E_ALL // NW
GCH = 120
BLK_E = 1600
BLK_N = 2000
INTERPRET = False


def _wid():
    return lax.axis_index("s") * NC + lax.axis_index("c")


def _mesh():
    return plsc.VectorSubcoreMesh(core_axis_name="c", subcore_axis_name="s",
                                  num_cores=NC, num_subcores=NS)


_SC_PARAMS = pltpu.CompilerParams(needs_layout_passes=False)


def _mo8(x):
    return pl.multiple_of(x, 8)


def _match_body(dstall, slist, mcnt, dstbuf, mbuf, sortbuf, histo, posb,
                cbuf):
    w = _wid()
    iota = lax.iota(I32, 16)
    node_base = w * RNG
    zeros = jnp.zeros((16,), I32)
    sentinel = zeros + (SCRAP << 19)

    def shuf(x, idx):
        dn = lax.GatherDimensionNumbers(offset_dims=(),
                                        collapsed_slice_dims=(0,),
                                        start_index_map=(0,))
        return lax.gather(x, idx[:, None], dn, (1,),
                          mode=lax.GatherScatterMode.PROMISE_IN_BOUNDS)

    def run_info(dl):
        prev = shuf(dl, jnp.maximum(iota - 1, 0))
        nxt = shuf(dl, jnp.minimum(iota + 1, 15))
        is_start = (dl != prev) | (iota == 0)
        is_end = (dl != nxt) | (iota == 15)
        run_start = plsc.cummax(jnp.where(is_start, iota, zeros))
        rank = iota - run_start
        return rank, is_end

    def per_type(t, _):
        base0 = (t * NW + w) * CAP

        def sort_flush(nv, G):
            for k in range(HB // 16):
                histo[pl.ds(k * 16, 16)] = zeros

            def hist(g, _):
                v = mbuf[pl.ds(g * 16, 16)]
                dl = lax.sort(v) >> 19
                rank, is_end = run_info(dl)
                plsc.addupdate_scatter(histo, [dl], rank + 1, mask=is_end)
                return 0

            lax.fori_loop(0, nv, hist, 0)
            carry = zeros
            for k in range(HB // 16):
                v = histo[pl.ds(k * 16, 16)]
                inc = plsc.cumsum(v)
                posb[pl.ds(k * 16, 16)] = carry + inc - v
                carry = carry + (zeros + inc[15])

            def perm(g, _):
                sv = lax.sort(mbuf[pl.ds(g * 16, 16)])
                dl = sv >> 19
                rank, is_end = run_info(dl)
                base = plsc.load_gather(posb, [dl])
                plsc.store_scatter(sortbuf, [base + rank], sv)
                plsc.addupdate_scatter(posb, [dl], rank + 1, mask=is_end)
                return 0

            lax.fori_loop(0, nv, perm, 0)

            def flush(k, _):
                pltpu.sync_copy(
                    sortbuf.at[pl.ds(k * 128, 128)],
                    slist.at[pl.ds(_mo8(base0 + G + k * 128), 128)])
                return 0

            lax.fori_loop(0, nv // 8, flush, 0)

        def per_chunk(c, carry):
            G, cntv = carry
            pltpu.sync_copy(
                dstall.at[pl.ds(_mo8(t * E_T + c * SCAN_CHUNK), SCAN_CHUNK)],
                dstbuf)

            def scan(i, cntv):
                idx = i * 16 + iota
                v = plsc.load_gather(dstbuf, [idx])
                dl = v - node_base
                mask = (dl >= 0) & (dl < RNG)
                eidv = t * E_T + c * SCAN_CHUNK + idx
                packed = (dl << 19) + eidv
                pos = cntv + plsc.cumsum(mask.astype(I32)) - 1
                plsc.store_scatter(mbuf, [pos], packed, mask=mask)
                return cntv + plsc.all_reduce_population_count(mask)

            cntv = lax.fori_loop(0, SCAN_CHUNK // 16, scan, cntv)
            cnt = jnp.max(cntv)

            def do_round(args):
                G, cntv = args
                sort_flush(ROUND // 16, G)
                rem = cnt - ROUND

                def shift(g, _):
                    mv = plsc.load_gather(mbuf, [ROUND + g * 16 + iota])
                    plsc.store_scatter(mbuf, [g * 16 + iota], mv)
                    return 0

                lax.fori_loop(0, (rem + 15) // 16, shift, 0)
                return G + ROUND, cntv - ROUND

            return lax.cond(cnt >= ROUND, do_round, lambda a: a, (G, cntv))

        G, cntv = lax.fori_loop(0, N_SCAN, per_chunk, (0, zeros))
        cnt = jnp.max(cntv)
        for k in range(128 // 16):
            plsc.store_scatter(mbuf, [cnt + k * 16 + iota], sentinel)
        cnt_pad = ((cnt + 127) // 128) * 128
        sort_flush(cnt_pad // 16, G)
        G = G + cnt_pad
        cbuf[...] = zeros + G
        pltpu.sync_copy(cbuf, mcnt.at[pl.ds(_mo8((t * NW + w) * 16), 16)])
        return 0

    lax.fori_loop(0, N_TYPES, per_type, 0)


def _match_call(dstall):
    return pl.kernel(
        _match_body,
        compiler_params=_SC_PARAMS,
        interpret=INTERPRET,
        out_type=(jax.ShapeDtypeStruct((N_TYPES * NW * CAP,), I32),
                  jax.ShapeDtypeStruct((N_TYPES * NW * 16,), I32)),
        mesh=_mesh(),
        scratch_types=[
            pltpu.VMEM((SCAN_CHUNK,), I32),
            pltpu.VMEM((MB_CAP,), I32),
            pltpu.VMEM((ROUND,), I32),
            pltpu.VMEM((HB,), I32),
            pltpu.VMEM((HB,), I32),
            pltpu.VMEM((16,), I32),
        ],
    )(dstall)


def _seg_body(msg, slist, mcnt, aggs, acc, pbuf, idxbuf, rows, cntbuf, sem):
    w = _wid()
    iota = lax.iota(I32, 16)
    neg = jnp.full((16,), -jnp.inf, F32)

    def per_type(t, _):
        def init_acc(i, _):
            plsc.store_scatter(acc, [i * 16 + iota], neg)
            return 0

        lax.fori_loop(0, (ACCROWS * H) // 16, init_acc, 0)
        pltpu.sync_copy(mcnt.at[pl.ds(_mo8((t * NW + w) * 16), 16)], cntbuf)
        cnt = jnp.max(cntbuf[...])
        base0 = (t * NW + w) * CAP

        def per_chunk(c, carry):
            pltpu.sync_copy(
                slist.at[pl.ds(_mo8(base0 + c * RMW_CHUNK), RMW_CHUNK)],
                pbuf)
            for i in range(RMW_CHUNK // 16):
                v = pbuf[pl.ds(i * 16, 16)]
                idxbuf[pl.ds(i * 16, 16)] = v & 0x7FFFF
            pltpu.async_copy(msg.at[idxbuf], rows, sem).wait()

            def group(g, carry):
                cur = carry[0]
                regs = list(carry[1:])
                v = pbuf[pl.ds(g * 16, 16)]
                for k in range(16):
                    dl = v[k] >> 19
                    fl = dl != cur
                    cur_old = cur
                    regs_old = tuple(regs)

                    @pl.when(fl)
                    def _():
                        for f in range(H // 16):
                            a = acc[pl.ds(cur_old * H + f * 16, 16)]
                            acc[pl.ds(cur_old * H + f * 16, 16)] = (
                                jnp.maximum(a, regs_old[f]))

                    j = g * 16 + k
                    for f in range(H // 16):
                        r = rows[j, pl.ds(f * 16, 16)]
                        regs[f] = jnp.where(fl, r, jnp.maximum(regs[f], r))
                    cur = jnp.where(fl, dl, cur)
                return (cur, *regs)

            return lax.fori_loop(0, RMW_CHUNK // 16, group, carry)

        init = (jnp.int32(SCRAP),) + (neg,) * (H // 16)
        fin = lax.fori_loop(0, cnt // RMW_CHUNK, per_chunk, init)
        cur = fin[0]
        for f in range(H // 16):
            a = acc[pl.ds(cur * H + f * 16, 16)]
            acc[pl.ds(cur * H + f * 16, 16)] = jnp.maximum(a, fin[1 + f])
        pltpu.sync_copy(
            acc.at[pl.ds(0, RNG * H)],
            aggs.at[pl.ds(_mo8((t * NPAD + w * RNG) * H), RNG * H)])
        return 0

    lax.fori_loop(0, N_TYPES, per_type, 0)


def _seg_call(msg, slist, mcnt):
    return pl.kernel(
        _seg_body,
        compiler_params=_SC_PARAMS,
        interpret=INTERPRET,
        out_type=jax.ShapeDtypeStruct((N_TYPES * NPAD * H,), F32),
        mesh=_mesh(),
        scratch_types=[
            pltpu.VMEM((ACCROWS * H,), F32),
            pltpu.VMEM((RMW_CHUNK,), I32),
            pltpu.VMEM((RMW_CHUNK,), I32),
            pltpu.VMEM((RMW_CHUNK, H), F32),
            pltpu.VMEM((16,), I32),
            pltpu.SemaphoreType.DMA,
        ],
    )(msg, slist, mcnt)


def _gather_body(ya, dstall, g, idxb, rows, sem):
    w = _wid()

    def per_chunk(c, _):
        base = _mo8(w * EPW + c * GCH)
        pltpu.sync_copy(dstall.at[pl.ds(base, GCH)], idxb)
        pltpu.async_copy(ya.at[idxb], rows, sem).wait()
        pltpu.sync_copy(rows, g.at[pl.ds(base, GCH)])
        return 0

    lax.fori_loop(0, EPW // GCH, per_chunk, 0)


def _gather_call(ya, dstall):
    return pl.kernel(
        _gather_body,
        compiler_params=_SC_PARAMS,
        interpret=INTERPRET,
        out_type=jax.ShapeDtypeStruct((E_ALL, H), F32),
        mesh=_mesh(),
        scratch_types=[
            pltpu.VMEM((GCH,), I32),
            pltpu.VMEM((GCH, H), F32),
            pltpu.SemaphoreType.DMA,
        ],
    )(ya, dstall)


def _dot(a, b):
    return jnp.dot(a, b, preferred_element_type=F32)


def _full(shape):
    return pl.BlockSpec(shape, lambda i: (0, 0))


def _agg_max(a0, a1, a2):
    m = lambda a: jnp.where(jnp.isneginf(a), 0.0, a)
    return jnp.maximum(jnp.maximum(m(a0), m(a1)), m(a2))


def _embed_kernel(x, w0, b0, w1, b1, o):
    h = jnp.maximum(_dot(x[...], w0[...]) + b0[...], 0.0)
    o[...] = _dot(h, w1[...]) + b1[...]


def _embed_call(x, w0, b0, w1, b1, blk):
    n, d = x.shape
    return pl.pallas_call(
        _embed_kernel,
        grid=(n // blk,),
        in_specs=[pl.BlockSpec((blk, d), lambda i: (i, 0)),
                  _full(w0.shape), _full(b0.shape),
                  _full(w1.shape), _full(b1.shape)],
        out_specs=pl.BlockSpec((blk, H), lambda i: (i, 0)),
        out_shape=jax.ShapeDtypeStruct((n, H), F32),
        interpret=INTERPRET,
    )(x, w0, b0, w1, b1)


def _edge0_kernel(x, w0, b0, w1, b1, fw0, fb0, fw1, fb1, ea, msg):
    h = jnp.maximum(_dot(x[...], w0[...]) + b0[...], 0.0)
    e = _dot(h, w1[...]) + b1[...]
    ea[...] = e
    h2 = jnp.maximum(_dot(e, fw0[...]) + fb0[...], 0.0)
    msg[...] = _dot(h2, fw1[...]) + fb1[...]


def _edge0_call(x, w0, b0, w1, b1, fw0, fb0, fw1, fb1, blk):
    n, d = x.shape
    return pl.pallas_call(
        _edge0_kernel,
        grid=(n // blk,),
        in_specs=[pl.BlockSpec((blk, d), lambda i: (i, 0)),
                  _full(w0.shape), _full(b0.shape),
                  _full(w1.shape), _full(b1.shape),
                  _full(fw0.shape), _full(fb0.shape),
                  _full(fw1.shape), _full(fb1.shape)],
        out_specs=[pl.BlockSpec((blk, H), lambda i: (i, 0)),
                   pl.BlockSpec((blk, H), lambda i: (i, 0))],
        out_shape=[jax.ShapeDtypeStruct((n, H), F32),
                   jax.ShapeDtypeStruct((n, H), F32)],
        interpret=INTERPRET,
    )(x, w0, b0, w1, b1, fw0, fb0, fw1, fb1)


def _node_kernel(xa, a0, a1, a2, w0b, b0, xan, ya):
    x = xa[...] + _agg_max(a0[...], a1[...], a2[...])
    xan[...] = x
    ya[...] = _dot(x, w0b[...]) + b0[...]


def _node_call(xa, a0, a1, a2, w0b, b0, blk):
    n = xa.shape[0]
    bs = pl.BlockSpec((blk, H), lambda i: (i, 0))
    return pl.pallas_call(
        _node_kernel,
        grid=(n // blk,),
        in_specs=[bs, bs, bs, bs, _full(w0b.shape), _full(b0.shape)],
        out_specs=[bs, bs],
        out_shape=[jax.ShapeDtypeStruct((n, H), F32),
                   jax.ShapeDtypeStruct((n, H), F32)],
        interpret=INTERPRET,
    )(xa, a0, a1, a2, w0b, b0)


def _edge_kernel(ea, g, w0a, w1, b1, fw0, fb0, fw1, fb1, ean, msg):
    e = ea[...]
    t = jnp.maximum(_dot(e, w0a[...]) + g[...], 0.0)
    e = e + _dot(t, w1[...]) + b1[...]
    ean[...] = e
    h = jnp.maximum(_dot(e, fw0[...]) + fb0[...], 0.0)
    msg[...] = _dot(h, fw1[...]) + fb1[...]


def _edge_call(ea, g, w0a, w1, b1, fw0, fb0, fw1, fb1, blk):
    n = ea.shape[0]
    bs = pl.BlockSpec((blk, H), lambda i: (i, 0))
    return pl.pallas_call(
        _edge_kernel,
        grid=(n // blk,),
        in_specs=[bs, bs,
                  _full(w0a.shape), _full(w1.shape), _full(b1.shape),
                  _full(fw0.shape), _full(fb0.shape),
                  _full(fw1.shape), _full(fb1.shape)],
        out_specs=[bs, bs],
        out_shape=[jax.ShapeDtypeStruct((n, H), F32),
                   jax.ShapeDtypeStruct((n, H), F32)],
        interpret=INTERPRET,
    )(ea, g, w0a, w1, b1, fw0, fb0, fw1, fb1)


def _field_kernel(xa, a0, a1, a2, act, w0x, w0a, b0, w1p, b1p, o):
    x = xa[...] + _agg_max(a0[...], a1[...], a2[...])
    h = jnp.maximum(_dot(x, w0x[...]) + _dot(act[...], w0a[...]) + b0[...],
                    0.0)
    o[...] = _dot(h, w1p[...]) + b1p[...]


def _field_call(xa, a0, a1, a2, act, w0x, w0a, b0, w1p, b1p, blk):
    n = xa.shape[0]
    bs = pl.BlockSpec((blk, H), lambda i: (i, 0))
    return pl.pallas_call(
        _field_kernel,
        grid=(n // blk,),
        in_specs=[bs, bs, bs, bs,
                  pl.BlockSpec((blk, ACT), lambda i: (i, 0)),
                  _full(w0x.shape), _full(w0a.shape), _full(b0.shape),
                  _full(w1p.shape), _full(b1p.shape)],
        out_specs=bs,
        out_shape=jax.ShapeDtypeStruct((n, H), F32),
        interpret=INTERPRET,
    )(xa, a0, a1, a2, act, w0x, w0a, b0, w1p, b1p)


def kernel(x_obstacle, x_agent, x_goal, edge_index_oa, edge_index_aa,
           edge_index_ga, edge_attr_oa, edge_attr_aa, edge_attr_ga, action,
           params):
    p = params
    r1 = lambda b: b.reshape(1, -1)

    dst_all = jnp.concatenate(
        [edge_index_oa[1], edge_index_aa[1], edge_index_ga[1]])
    ecat = jnp.concatenate([edge_attr_oa, edge_attr_aa, edge_attr_ga], axis=0)

    xa = _embed_call(x_agent, p["embed_W0"], r1(p["embed_b0"]),
                     p["embed_W1"], r1(p["embed_b1"]), BLK_N)

    ea, msg = _edge0_call(ecat, p["eembed_W0"], r1(p["eembed_b0"]),
                          p["eembed_W1"], r1(p["eembed_b1"]),
                          p["fx0_W0"], r1(p["fx0_b0"]),
                          p["fx0_W1"], r1(p["fx0_b1"]), BLK_E)

    slist, mcnt = _match_call(dst_all)

    for l in range(3):
        aggs = _seg_call(msg, slist, mcnt).reshape(N_TYPES, NPAD, H)
        a0, a1, a2 = (aggs[t][:N_NODE] for t in range(N_TYPES))
        if l == 2:
            w0 = p["field_W0"]
            w1p = jnp.pad(p["field_W1"], ((0, 0), (0, H - 1)))
            b1p = jnp.broadcast_to(p["field_b1"].reshape(1, 1), (1, H))
            out = _field_call(xa, a0, a1, a2, action, w0[:H], w0[H:],
                              r1(p["field_b0"]), w1p, b1p, BLK_N)
            return out[:, 0]
        em = "em%d_" % l
        w0 = p[em + "W0"]
        xa, ya = _node_call(xa, a0, a1, a2, w0[H:], r1(p[em + "b0"]), BLK_N)
        g = _gather_call(ya, dst_all)
        fx = "fx%d_" % (l + 1)
        ea, msg = _edge_call(ea, g, w0[:H], p[em + "W1"], r1(p[em + "b1"]),
                             p[fx + "W0"], r1(p[fx + "b0"]),
                             p[fx + "W1"], r1(p[fx + "b1"]), BLK_E)


# speedup vs baseline: 19.8713x; 1.3781x over previous
import functools

import jax
import jax.numpy as jnp
from jax import lax
from jax.experimental import pallas as pl
from jax.experimental.pallas import tpu as pltpu
from jax.experimental.pallas import tpu_sc as plsc

F32 = jnp.float32
BF16 = jnp.bfloat16
I32 = jnp.int32

N_NODE = 10000
E_T = 160000
N_TYPES = 3
E_ALL = N_TYPES * E_T
H = 128
ACT = 8

NC = 2
NS = 16
NW = NC * NS

RNG = 320
NPAD = NW * RNG
ACCROWS = RNG + 1
SCRAP = RNG
SCAN_CHUNK = 3200
N_SCAN = E_T // SCAN_CHUNK
ROUND = 16384
MB_CAP = ROUND + SCAN_CHUNK + 160
HB = 336
CAP = 160384
RMW_CHUNK = 128
EPW = E_ALL // NW
GCH = 120
BLK_E = 3200
BLK_N = 2000
INTERPRET = False


def _wid():
    return lax.axis_index("s") * NC + lax.axis_index("c")


def _mesh():
    return plsc.VectorSubcoreMesh(core_axis_name="c", subcore_axis_name="s",
                                  num_cores=NC, num_subcores=NS)


_SC_PARAMS = pltpu.CompilerParams(needs_layout_passes=False)


def _mo8(x):
    return pl.multiple_of(x, 8)


def _match_body(dstall, slist, mcnt, dstbuf, mbuf, sortbuf, histo, posb,
                cbuf):
    w = _wid()
    iota = lax.iota(I32, 16)
    node_base = w * RNG
    zeros = jnp.zeros((16,), I32)
    sentinel = zeros + (SCRAP << 19)

    def shuf(x, idx):
        dn = lax.GatherDimensionNumbers(offset_dims=(),
                                        collapsed_slice_dims=(0,),
                                        start_index_map=(0,))
        return lax.gather(x, idx[:, None], dn, (1,),
                          mode=lax.GatherScatterMode.PROMISE_IN_BOUNDS)

    def run_info(dl):
        prev = shuf(dl, jnp.maximum(iota - 1, 0))
        nxt = shuf(dl, jnp.minimum(iota + 1, 15))
        is_start = (dl != prev) | (iota == 0)
        is_end = (dl != nxt) | (iota == 15)
        run_start = plsc.cummax(jnp.where(is_start, iota, zeros))
        rank = iota - run_start
        return rank, is_end

    def per_type(t, _):
        base0 = (t * NW + w) * CAP

        def sort_flush(nv, G):
            for k in range(HB // 16):
                histo[pl.ds(k * 16, 16)] = zeros

            def hist(g, _):
                v = mbuf[pl.ds(g * 16, 16)]
                dl = lax.sort(v) >> 19
                rank, is_end = run_info(dl)
                plsc.addupdate_scatter(histo, [dl], rank + 1, mask=is_end)
                return 0

            lax.fori_loop(0, nv, hist, 0)
            carry = zeros
            for k in range(HB // 16):
                v = histo[pl.ds(k * 16, 16)]
                inc = plsc.cumsum(v)
                posb[pl.ds(k * 16, 16)] = carry + inc - v
                carry = carry + (zeros + inc[15])

            def perm(g, _):
                sv = lax.sort(mbuf[pl.ds(g * 16, 16)])
                dl = sv >> 19
                rank, is_end = run_info(dl)
                base = plsc.load_gather(posb, [dl])
                plsc.store_scatter(sortbuf, [base + rank], sv)
                plsc.addupdate_scatter(posb, [dl], rank + 1, mask=is_end)
                return 0

            lax.fori_loop(0, nv, perm, 0)

            def flush(k, _):
                pltpu.sync_copy(
                    sortbuf.at[pl.ds(k * 128, 128)],
                    slist.at[pl.ds(_mo8(base0 + G + k * 128), 128)])
                return 0

            lax.fori_loop(0, nv // 8, flush, 0)

        def per_chunk(c, carry):
            G, cntv = carry
            pltpu.sync_copy(
                dstall.at[pl.ds(_mo8(t * E_T + c * SCAN_CHUNK), SCAN_CHUNK)],
                dstbuf)

            def scan(i, cntv):
                vecs = []
                for u in range(4):
                    idx = i * 64 + u * 16 + iota
                    v = plsc.load_gather(dstbuf, [idx])
                    dl = v - node_base
                    mask = (dl >= 0) & (dl < RNG)
                    eidv = t * E_T + c * SCAN_CHUNK + idx
                    packed = (dl << 19) + eidv
                    cs = plsc.cumsum(mask.astype(I32))
                    pc = plsc.all_reduce_population_count(mask)
                    vecs.append((packed, mask, cs, pc))
                for packed, mask, cs, pc in vecs:
                    plsc.store_scatter(mbuf, [cntv + cs - 1], packed,
                                       mask=mask)
                    cntv = cntv + pc
                return cntv

            cntv = lax.fori_loop(0, SCAN_CHUNK // 64, scan, cntv)
            cnt = jnp.max(cntv)

            def do_round(args):
                G, cntv = args
                sort_flush(ROUND // 16, G)
                rem = cnt - ROUND

                def shift(g, _):
                    mv = plsc.load_gather(mbuf, [ROUND + g * 16 + iota])
                    plsc.store_scatter(mbuf, [g * 16 + iota], mv)
                    return 0

                lax.fori_loop(0, (rem + 15) // 16, shift, 0)
                return G + ROUND, cntv - ROUND

            return lax.cond(cnt >= ROUND, do_round, lambda a: a, (G, cntv))

        G, cntv = lax.fori_loop(0, N_SCAN, per_chunk, (0, zeros))
        cnt = jnp.max(cntv)
        for k in range(128 // 16):
            plsc.store_scatter(mbuf, [cnt + k * 16 + iota], sentinel)
        cnt_pad = ((cnt + 127) // 128) * 128
        sort_flush(cnt_pad // 16, G)
        G = G + cnt_pad
        cbuf[...] = zeros + G
        pltpu.sync_copy(cbuf, mcnt.at[pl.ds(_mo8((t * NW + w) * 16), 16)])
        return 0

    lax.fori_loop(0, N_TYPES, per_type, 0)


def _match_call(dstall):
    return pl.kernel(
        _match_body,
        compiler_params=_SC_PARAMS,
        interpret=INTERPRET,
        out_type=(jax.ShapeDtypeStruct((N_TYPES * NW * CAP,), I32),
                  jax.ShapeDtypeStruct((N_TYPES * NW * 16,), I32)),
        mesh=_mesh(),
        scratch_types=[
            pltpu.VMEM((SCAN_CHUNK,), I32),
            pltpu.VMEM((MB_CAP,), I32),
            pltpu.VMEM((ROUND,), I32),
            pltpu.VMEM((HB,), I32),
            pltpu.VMEM((HB,), I32),
            pltpu.VMEM((16,), I32),
        ],
    )(dstall)


def _seg_body(msg, slist, mcnt, aggs, acc, pbulk, pbuf, idxbuf, rows,
              cntbuf, sem):
    w = _wid()
    neg = jnp.full((16,), -jnp.inf, F32)
    BULK = 32768

    def _off(cur, f):
        return cur * H + f * 16

    def per_type(t, _):
        def init_acc(i, _):
            acc[pl.ds(i * 16, 16)] = neg
            return 0

        lax.fori_loop(0, (ACCROWS * H) // 16, init_acc, 0)
        pltpu.sync_copy(mcnt.at[pl.ds(_mo8((t * NW + w) * 16), 16)], cntbuf)
        cnt = jnp.max(cntbuf[...])
        base0 = (t * NW + w) * CAP
        nch = cnt // RMW_CHUNK
        init = (jnp.int32(SCRAP),) + (neg,) * 8

        def mk_loop(src, src_off, pre_stage):
            def stage(c, off):
                for i in range(RMW_CHUNK // 16):
                    v = src[pl.ds(src_off(c) + i * 16, 16)]
                    idxbuf[pl.ds(off + i * 16, 16)] = v & 0x7FFFF
                pltpu.async_copy(
                    msg.at[idxbuf.at[pl.ds(off, RMW_CHUNK)]],
                    rows.at[pl.ds(_mo8(off), RMW_CHUNK)], sem)

            @pl.when(nch > 0)
            def _():
                pre_stage(0)
                stage(0, 0)

            def per_chunk(c, carry):
                poff = (c & 1) * RMW_CHUNK
                qoff = RMW_CHUNK - poff
                pltpu.make_async_copy(
                    msg.at[idxbuf.at[pl.ds(poff, RMW_CHUNK)]],
                    rows.at[pl.ds(_mo8(poff), RMW_CHUNK)], sem).wait()

                @pl.when(c + 1 < nch)
                def _():
                    pre_stage(c + 1)
                    stage(c + 1, qoff)

                def group(g, carry):
                    cur = carry[0]
                    regs = list(carry[1:])
                    v = src[pl.ds(src_off(c) + g * 16, 16)]
                    for k in range(16):
                        dl = v[k] >> 19
                        fl = dl != cur
                        cur_old = cur
                        regs_old = tuple(regs)

                        @pl.when(fl)
                        def _():
                            for f in range(8):
                                a = acc[pl.ds(_off(cur_old, f), 16)]
                                acc[pl.ds(_off(cur_old, f), 16)] = (
                                    jnp.maximum(a, regs_old[f]))

                        j = poff + g * 16 + k
                        for f in range(8):
                            r = rows[j, pl.ds(f * 16, 16)]
                            regs[f] = jnp.where(fl, r,
                                                jnp.maximum(regs[f], r))
                        cur = jnp.where(fl, dl, cur)
                    return (cur, *regs)

                return lax.fori_loop(0, RMW_CHUNK // 16, group, carry)

            return lax.fori_loop(0, nch, per_chunk, init)

        def fast(_):
            def load2k(k, _):
                pltpu.sync_copy(
                    slist.at[pl.ds(_mo8(base0 + k * 2048), 2048)],
                    pbulk.at[pl.ds(k * 2048, 2048)])
                return 0

            lax.fori_loop(0, cnt // 2048, load2k, 0)

            def load128(k, _):
                pltpu.sync_copy(
                    slist.at[pl.ds(_mo8(base0 + k * 128), 128)],
                    pbulk.at[pl.ds(k * 128, 128)])
                return 0

            lax.fori_loop((cnt // 2048) * 16, nch, load128, 0)
            return mk_loop(pbulk, lambda c: c * RMW_CHUNK, lambda c: None)

        def slow(_):
            def pre_stage(c):
                pltpu.sync_copy(
                    slist.at[pl.ds(_mo8(base0 + c * RMW_CHUNK), RMW_CHUNK)],
                    pbuf.at[pl.ds((c & 1) * RMW_CHUNK, RMW_CHUNK)])

            return mk_loop(pbuf, lambda c: (c & 1) * RMW_CHUNK, pre_stage)

        fin = lax.cond(cnt <= BULK, fast, slow, 0)
        cur = fin[0]
        for f in range(8):
            a = acc[pl.ds(_off(cur, f), 16)]
            acc[pl.ds(_off(cur, f), 16)] = jnp.maximum(a, fin[1 + f])
        pltpu.sync_copy(
            acc.at[pl.ds(0, RNG * H)],
            aggs.at[pl.ds(_mo8((t * NPAD + w * RNG) * H), RNG * H)])
        return 0

    lax.fori_loop(0, N_TYPES, per_type, 0)


def _seg_call(msg, slist, mcnt):
    return pl.kernel(
        _seg_body,
        compiler_params=_SC_PARAMS,
        interpret=INTERPRET,
        out_type=jax.ShapeDtypeStruct((N_TYPES * NPAD * H,), F32),
        mesh=_mesh(),
        scratch_types=[
            pltpu.VMEM((ACCROWS * H,), F32),
            pltpu.VMEM((32768,), I32),
            pltpu.VMEM((2 * RMW_CHUNK,), I32),
            pltpu.VMEM((2 * RMW_CHUNK,), I32),
            pltpu.VMEM((2 * RMW_CHUNK, H), F32),
            pltpu.VMEM((16,), I32),
            pltpu.SemaphoreType.DMA,
        ],
    )(msg, slist, mcnt)


def _gather_body(ya, dstall, g, idxb, rows, sem):
    w = _wid()
    nch = EPW // GCH
    pltpu.sync_copy(dstall.at[pl.ds(_mo8(w * EPW), EPW)], idxb)

    def fire(c, off):
        pltpu.async_copy(ya.at[idxb.at[pl.ds(_mo8(c * GCH), GCH)]],
                         rows.at[pl.ds(_mo8(off), GCH)], sem)

    fire(0, 0)

    def per_chunk(c, _):
        poff = (c & 1) * GCH
        qoff = GCH - poff
        pltpu.make_async_copy(ya.at[idxb.at[pl.ds(_mo8(c * GCH), GCH)]],
                              rows.at[pl.ds(_mo8(poff), GCH)], sem).wait()

        @pl.when(c + 1 < nch)
        def _():
            fire(c + 1, qoff)

        pltpu.sync_copy(rows.at[pl.ds(_mo8(poff), GCH)],
                        g.at[pl.ds(_mo8(w * EPW + c * GCH), GCH)])
        return 0

    lax.fori_loop(0, nch, per_chunk, 0)


def _gather_call(ya, dstall):
    return pl.kernel(
        _gather_body,
        compiler_params=_SC_PARAMS,
        interpret=INTERPRET,
        out_type=jax.ShapeDtypeStruct((E_ALL, H), F32),
        mesh=_mesh(),
        scratch_types=[
            pltpu.VMEM((EPW,), I32),
            pltpu.VMEM((2 * GCH, H), F32),
            pltpu.SemaphoreType.DMA,
        ],
    )(ya, dstall)


def _dot(a, b):
    return jnp.dot(a, b, preferred_element_type=F32)


def _full(shape):
    return pl.BlockSpec(shape, lambda i: (0, 0))


def _agg_max(a0, a1, a2):
    m = lambda a: jnp.where(jnp.isneginf(a), 0.0, a)
    return jnp.maximum(jnp.maximum(m(a0), m(a1)), m(a2))


def _embed_kernel(x, w0, b0, w1, b1, o):
    h = jnp.maximum(_dot(x[...], w0[...]) + b0[...], 0.0)
    o[...] = _dot(h, w1[...]) + b1[...]


def _embed_call(x, w0, b0, w1, b1, blk):
    n, d = x.shape
    return pl.pallas_call(
        _embed_kernel,
        grid=(n // blk,),
        in_specs=[pl.BlockSpec((blk, d), lambda i: (i, 0)),
                  _full(w0.shape), _full(b0.shape),
                  _full(w1.shape), _full(b1.shape)],
        out_specs=pl.BlockSpec((blk, H), lambda i: (i, 0)),
        out_shape=jax.ShapeDtypeStruct((n, H), F32),
        interpret=INTERPRET,
    )(x, w0, b0, w1, b1)


def _edge0_kernel(x, w0, b0, w1, b1, fw0, fb0, fw1, fb1, ea, msg):
    h = jnp.maximum(_dot(x[...], w0[...]) + b0[...], 0.0)
    e = _dot(h, w1[...]) + b1[...]
    ea[...] = e
    h2 = jnp.maximum(_dot(e, fw0[...]) + fb0[...], 0.0)
    msg[...] = _dot(h2, fw1[...]) + fb1[...]


def _edge0_call(x, w0, b0, w1, b1, fw0, fb0, fw1, fb1, blk):
    n, d = x.shape
    return pl.pallas_call(
        _edge0_kernel,
        grid=(n // blk,),
        in_specs=[pl.BlockSpec((blk, d), lambda i: (i, 0)),
                  _full(w0.shape), _full(b0.shape),
                  _full(w1.shape), _full(b1.shape),
                  _full(fw0.shape), _full(fb0.shape),
                  _full(fw1.shape), _full(fb1.shape)],
        out_specs=[pl.BlockSpec((blk, H), lambda i: (i, 0)),
                   pl.BlockSpec((blk, H), lambda i: (i, 0))],
        out_shape=[jax.ShapeDtypeStruct((n, H), F32),
                   jax.ShapeDtypeStruct((n, H), F32)],
        interpret=INTERPRET,
    )(x, w0, b0, w1, b1, fw0, fb0, fw1, fb1)


def _node_kernel(xa, a0, a1, a2, w0b, b0, xan, ya):
    x = xa[...] + _agg_max(a0[...], a1[...], a2[...])
    xan[...] = x
    ya[...] = _dot(x, w0b[...]) + b0[...]


def _node_call(xa, a0, a1, a2, w0b, b0, blk):
    n = xa.shape[0]
    bs = pl.BlockSpec((blk, H), lambda i: (i, 0))
    return pl.pallas_call(
        _node_kernel,
        grid=(n // blk,),
        in_specs=[bs, bs, bs, bs, _full(w0b.shape), _full(b0.shape)],
        out_specs=[bs, bs],
        out_shape=[jax.ShapeDtypeStruct((n, H), F32),
                   jax.ShapeDtypeStruct((n, H), F32)],
        interpret=INTERPRET,
    )(xa, a0, a1, a2, w0b, b0)


def _edge_kernel(ea, g, w0a, w1, b1, fw0, fb0, fw1, fb1, ean, msg):
    e = ea[...]
    t = jnp.maximum(_dot(e, w0a[...]) + g[...], 0.0)
    e = e + _dot(t, w1[...]) + b1[...]
    ean[...] = e
    h = jnp.maximum(_dot(e, fw0[...]) + fb0[...], 0.0)
    msg[...] = _dot(h, fw1[...]) + fb1[...]


def _edge_call(ea, g, w0a, w1, b1, fw0, fb0, fw1, fb1, blk):
    n = ea.shape[0]
    bs = pl.BlockSpec((blk, H), lambda i: (i, 0))
    return pl.pallas_call(
        _edge_kernel,
        grid=(n // blk,),
        in_specs=[bs, bs,
                  _full(w0a.shape), _full(w1.shape), _full(b1.shape),
                  _full(fw0.shape), _full(fb0.shape),
                  _full(fw1.shape), _full(fb1.shape)],
        out_specs=[bs, bs],
        out_shape=[jax.ShapeDtypeStruct((n, H), F32),
                   jax.ShapeDtypeStruct((n, H), F32)],
        interpret=INTERPRET,
    )(ea, g, w0a, w1, b1, fw0, fb0, fw1, fb1)


def _field_kernel(xa, a0, a1, a2, act, w0x, w0a, b0, w1p, b1p, o):
    x = xa[...] + _agg_max(a0[...], a1[...], a2[...])
    h = jnp.maximum(_dot(x, w0x[...]) + _dot(act[...], w0a[...]) + b0[...],
                    0.0)
    o[...] = _dot(h, w1p[...]) + b1p[...]


def _field_call(xa, a0, a1, a2, act, w0x, w0a, b0, w1p, b1p, blk):
    n = xa.shape[0]
    bs = pl.BlockSpec((blk, H), lambda i: (i, 0))
    return pl.pallas_call(
        _field_kernel,
        grid=(n // blk,),
        in_specs=[bs, bs, bs, bs,
                  pl.BlockSpec((blk, ACT), lambda i: (i, 0)),
                  _full(w0x.shape), _full(w0a.shape), _full(b0.shape),
                  _full(w1p.shape), _full(b1p.shape)],
        out_specs=bs,
        out_shape=jax.ShapeDtypeStruct((n, H), F32),
        interpret=INTERPRET,
    )(xa, a0, a1, a2, act, w0x, w0a, b0, w1p, b1p)


def kernel(x_obstacle, x_agent, x_goal, edge_index_oa, edge_index_aa,
           edge_index_ga, edge_attr_oa, edge_attr_aa, edge_attr_ga, action,
           params):
    p = params
    r1 = lambda b: b.reshape(1, -1)

    dst_all = jnp.concatenate(
        [edge_index_oa[1], edge_index_aa[1], edge_index_ga[1]])
    ecat = jnp.concatenate([edge_attr_oa, edge_attr_aa, edge_attr_ga], axis=0)

    xa = _embed_call(x_agent, p["embed_W0"], r1(p["embed_b0"]),
                     p["embed_W1"], r1(p["embed_b1"]), BLK_N)

    ea, msg = _edge0_call(ecat, p["eembed_W0"], r1(p["eembed_b0"]),
                          p["eembed_W1"], r1(p["eembed_b1"]),
                          p["fx0_W0"], r1(p["fx0_b0"]),
                          p["fx0_W1"], r1(p["fx0_b1"]), BLK_E)

    slist, mcnt = _match_call(dst_all)

    for l in range(3):
        aggs = _seg_call(msg, slist, mcnt).reshape(N_TYPES, NPAD, H)
        a0, a1, a2 = (aggs[t][:N_NODE] for t in range(N_TYPES))
        if l == 2:
            w0 = p["field_W0"]
            w1p = jnp.pad(p["field_W1"], ((0, 0), (0, H - 1)))
            b1p = jnp.broadcast_to(p["field_b1"].reshape(1, 1), (1, H))
            out = _field_call(xa, a0, a1, a2, action, w0[:H], w0[H:],
                              r1(p["field_b0"]), w1p, b1p, BLK_N)
            return out[:, 0]
        em = "em%d_" % l
        w0 = p[em + "W0"]
        xa, ya = _node_call(xa, a0, a1, a2, w0[H:], r1(p[em + "b0"]), BLK_N)
        g = _gather_call(ya, dst_all)
        fx = "fx%d_" % (l + 1)
        ea, msg = _edge_call(ea, g, w0[:H], p[em + "W1"], r1(p[em + "b1"]),
                             p[fx + "W0"], r1(p[fx + "b0"]),
                             p[fx + "W1"], r1(p[fx + "b1"]), BLK_E)


# speedup vs baseline: 20.3713x; 1.0252x over previous
import functools

import jax
import jax.numpy as jnp
from jax import lax
from jax.experimental import pallas as pl
from jax.experimental.pallas import tpu as pltpu
from jax.experimental.pallas import tpu_sc as plsc

F32 = jnp.float32
BF16 = jnp.bfloat16
I32 = jnp.int32

N_NODE = 10000
E_T = 160000
N_TYPES = 3
E_ALL = N_TYPES * E_T
H = 128
ACT = 8

NC = 2
NS = 16
NW = NC * NS

RNG = 320
NPAD = NW * RNG
ACCROWS = RNG + 1
SCRAP = RNG
SCAN_CHUNK = 3200
N_SCAN = E_T // SCAN_CHUNK
ROUND = 16384
MB_CAP = ROUND + SCAN_CHUNK + 160
HB = 336
CAP = 160384
RMW_CHUNK = 128
EPW = E_ALL // NW
GCH = 120
BLK_E = 4800
BLK_N = 5000
INTERPRET = False


def _wid():
    return lax.axis_index("s") * NC + lax.axis_index("c")


def _mesh():
    return plsc.VectorSubcoreMesh(core_axis_name="c", subcore_axis_name="s",
                                  num_cores=NC, num_subcores=NS)


_SC_PARAMS = pltpu.CompilerParams(needs_layout_passes=False)


def _mo8(x):
    return pl.multiple_of(x, 8)


def _match_body(dstall, slist, mcnt, dstbuf, mbuf, sortbuf, histo, posb,
                cbuf):
    w = _wid()
    iota = lax.iota(I32, 16)
    node_base = w * RNG
    zeros = jnp.zeros((16,), I32)
    sentinel = zeros + (SCRAP << 19)

    def shuf(x, idx):
        dn = lax.GatherDimensionNumbers(offset_dims=(),
                                        collapsed_slice_dims=(0,),
                                        start_index_map=(0,))
        return lax.gather(x, idx[:, None], dn, (1,),
                          mode=lax.GatherScatterMode.PROMISE_IN_BOUNDS)

    def run_info(dl):
        prev = shuf(dl, jnp.maximum(iota - 1, 0))
        nxt = shuf(dl, jnp.minimum(iota + 1, 15))
        is_start = (dl != prev) | (iota == 0)
        is_end = (dl != nxt) | (iota == 15)
        run_start = plsc.cummax(jnp.where(is_start, iota, zeros))
        rank = iota - run_start
        return rank, is_end

    def per_type(t, _):
        base0 = (t * NW + w) * CAP

        def sort_flush(nv, G):
            for k in range(HB // 16):
                histo[pl.ds(k * 16, 16)] = zeros

            def hist(g, _):
                v = mbuf[pl.ds(g * 16, 16)]
                dl = lax.sort(v) >> 19
                rank, is_end = run_info(dl)
                plsc.addupdate_scatter(histo, [dl], rank + 1, mask=is_end)
                return 0

            lax.fori_loop(0, nv, hist, 0)
            carry = zeros
            for k in range(HB // 16):
                v = histo[pl.ds(k * 16, 16)]
                inc = plsc.cumsum(v)
                posb[pl.ds(k * 16, 16)] = carry + inc - v
                carry = carry + (zeros + inc[15])

            def perm(g, _):
                sv = lax.sort(mbuf[pl.ds(g * 16, 16)])
                dl = sv >> 19
                rank, is_end = run_info(dl)
                base = plsc.load_gather(posb, [dl])
                plsc.store_scatter(sortbuf, [base + rank], sv)
                plsc.addupdate_scatter(posb, [dl], rank + 1, mask=is_end)
                return 0

            lax.fori_loop(0, nv, perm, 0)

            def flush(k, _):
                pltpu.sync_copy(
                    sortbuf.at[pl.ds(k * 128, 128)],
                    slist.at[pl.ds(_mo8(base0 + G + k * 128), 128)])
                return 0

            lax.fori_loop(0, nv // 8, flush, 0)

        def per_chunk(c, carry):
            G, cntv = carry
            pltpu.sync_copy(
                dstall.at[pl.ds(_mo8(t * E_T + c * SCAN_CHUNK), SCAN_CHUNK)],
                dstbuf)

            def scan(i, cntv):
                vecs = []
                for u in range(4):
                    idx = i * 64 + u * 16 + iota
                    v = plsc.load_gather(dstbuf, [idx])
                    dl = v - node_base
                    mask = (dl >= 0) & (dl < RNG)
                    eidv = t * E_T + c * SCAN_CHUNK + idx
                    packed = (dl << 19) + eidv
                    cs = plsc.cumsum(mask.astype(I32))
                    pc = plsc.all_reduce_population_count(mask)
                    vecs.append((packed, mask, cs, pc))
                for packed, mask, cs, pc in vecs:
                    plsc.store_scatter(mbuf, [cntv + cs - 1], packed,
                                       mask=mask)
                    cntv = cntv + pc
                return cntv

            cntv = lax.fori_loop(0, SCAN_CHUNK // 64, scan, cntv)
            cnt = jnp.max(cntv)

            def do_round(args):
                G, cntv = args
                sort_flush(ROUND // 16, G)
                rem = cnt - ROUND

                def shift(g, _):
                    mv = plsc.load_gather(mbuf, [ROUND + g * 16 + iota])
                    plsc.store_scatter(mbuf, [g * 16 + iota], mv)
                    return 0

                lax.fori_loop(0, (rem + 15) // 16, shift, 0)
                return G + ROUND, cntv - ROUND

            return lax.cond(cnt >= ROUND, do_round, lambda a: a, (G, cntv))

        G, cntv = lax.fori_loop(0, N_SCAN, per_chunk, (0, zeros))
        cnt = jnp.max(cntv)
        for k in range(128 // 16):
            plsc.store_scatter(mbuf, [cnt + k * 16 + iota], sentinel)
        cnt_pad = ((cnt + 127) // 128) * 128
        sort_flush(cnt_pad // 16, G)
        G = G + cnt_pad
        cbuf[...] = zeros + G
        pltpu.sync_copy(cbuf, mcnt.at[pl.ds(_mo8((t * NW + w) * 16), 16)])
        return 0

    lax.fori_loop(0, N_TYPES, per_type, 0)


def _match_call(dstall):
    return pl.kernel(
        _match_body,
        compiler_params=_SC_PARAMS,
        interpret=INTERPRET,
        out_type=(jax.ShapeDtypeStruct((N_TYPES * NW * CAP,), I32),
                  jax.ShapeDtypeStruct((N_TYPES * NW * 16,), I32)),
        mesh=_mesh(),
        scratch_types=[
            pltpu.VMEM((SCAN_CHUNK,), I32),
            pltpu.VMEM((MB_CAP,), I32),
            pltpu.VMEM((ROUND,), I32),
            pltpu.VMEM((HB,), I32),
            pltpu.VMEM((HB,), I32),
            pltpu.VMEM((16,), I32),
        ],
    )(dstall)


def _seg_body(msg, slist, mcnt, aggs, acc, pbulk, pbuf, idxbuf, rows,
              cntbuf, sem):
    w = _wid()
    neg = jnp.full((16,), -jnp.inf, F32)
    BULK = 32768

    def _off(cur, f):
        return cur * H + f * 16

    def per_type(t, _):
        def init_acc(i, _):
            acc[pl.ds(i * 16, 16)] = neg
            return 0

        lax.fori_loop(0, (ACCROWS * H) // 16, init_acc, 0)
        pltpu.sync_copy(mcnt.at[pl.ds(_mo8((t * NW + w) * 16), 16)], cntbuf)
        cnt = jnp.max(cntbuf[...])
        base0 = (t * NW + w) * CAP
        nch = cnt // RMW_CHUNK
        init = (jnp.int32(SCRAP),) + (neg,) * 8

        def mk_loop(src, src_off, pre_stage):
            def stage(c, off):
                for i in range(RMW_CHUNK // 16):
                    v = src[pl.ds(src_off(c) + i * 16, 16)]
                    idxbuf[pl.ds(off + i * 16, 16)] = v & 0x7FFFF
                pltpu.async_copy(
                    msg.at[idxbuf.at[pl.ds(off, RMW_CHUNK)]],
                    rows.at[pl.ds(_mo8(off), RMW_CHUNK)], sem)

            @pl.when(nch > 0)
            def _():
                pre_stage(0)
                stage(0, 0)

            def per_chunk(c, carry):
                poff = (c & 1) * RMW_CHUNK
                qoff = RMW_CHUNK - poff
                pltpu.make_async_copy(
                    msg.at[idxbuf.at[pl.ds(poff, RMW_CHUNK)]],
                    rows.at[pl.ds(_mo8(poff), RMW_CHUNK)], sem).wait()

                @pl.when(c + 1 < nch)
                def _():
                    pre_stage(c + 1)
                    stage(c + 1, qoff)

                def group(g, carry):
                    cur = carry[0]
                    regs = list(carry[1:])
                    v = src[pl.ds(src_off(c) + g * 16, 16)]
                    for k in range(16):
                        dl = v[k] >> 19
                        fl = dl != cur
                        cur_old = cur
                        regs_old = tuple(regs)

                        @pl.when(fl)
                        def _():
                            for f in range(8):
                                a = acc[pl.ds(_off(cur_old, f), 16)]
                                acc[pl.ds(_off(cur_old, f), 16)] = (
                                    jnp.maximum(a, regs_old[f]))

                        j = poff + g * 16 + k
                        for f in range(8):
                            r = rows[j, pl.ds(f * 16, 16)]
                            regs[f] = jnp.where(fl, r,
                                                jnp.maximum(regs[f], r))
                        cur = jnp.where(fl, dl, cur)
                    return (cur, *regs)

                return lax.fori_loop(0, RMW_CHUNK // 16, group, carry)

            return lax.fori_loop(0, nch, per_chunk, init)

        def fast(_):
            def load2k(k, _):
                pltpu.sync_copy(
                    slist.at[pl.ds(_mo8(base0 + k * 2048), 2048)],
                    pbulk.at[pl.ds(k * 2048, 2048)])
                return 0

            lax.fori_loop(0, cnt // 2048, load2k, 0)

            def load128(k, _):
                pltpu.sync_copy(
                    slist.at[pl.ds(_mo8(base0 + k * 128), 128)],
                    pbulk.at[pl.ds(k * 128, 128)])
                return 0

            lax.fori_loop((cnt // 2048) * 16, nch, load128, 0)
            return mk_loop(pbulk, lambda c: c * RMW_CHUNK, lambda c: None)

        def slow(_):
            def pre_stage(c):
                pltpu.sync_copy(
                    slist.at[pl.ds(_mo8(base0 + c * RMW_CHUNK), RMW_CHUNK)],
                    pbuf.at[pl.ds((c & 1) * RMW_CHUNK, RMW_CHUNK)])

            return mk_loop(pbuf, lambda c: (c & 1) * RMW_CHUNK, pre_stage)

        fin = lax.cond(cnt <= BULK, fast, slow, 0)
        cur = fin[0]
        for f in range(8):
            a = acc[pl.ds(_off(cur, f), 16)]
            acc[pl.ds(_off(cur, f), 16)] = jnp.maximum(a, fin[1 + f])
        pltpu.sync_copy(
            acc.at[pl.ds(0, RNG * H)],
            aggs.at[pl.ds(_mo8((t * NPAD + w * RNG) * H), RNG * H)])
        return 0

    lax.fori_loop(0, N_TYPES, per_type, 0)


def _seg_call(msg, slist, mcnt):
    return pl.kernel(
        _seg_body,
        compiler_params=_SC_PARAMS,
        interpret=INTERPRET,
        out_type=jax.ShapeDtypeStruct((N_TYPES * NPAD * H,), F32),
        mesh=_mesh(),
        scratch_types=[
            pltpu.VMEM((ACCROWS * H,), F32),
            pltpu.VMEM((32768,), I32),
            pltpu.VMEM((2 * RMW_CHUNK,), I32),
            pltpu.VMEM((2 * RMW_CHUNK,), I32),
            pltpu.VMEM((2 * RMW_CHUNK, H), F32),
            pltpu.VMEM((16,), I32),
            pltpu.SemaphoreType.DMA,
        ],
    )(msg, slist, mcnt)


def _gather_body(ya, dstall, g, idxb, rows, sem):
    w = _wid()
    nch = EPW // GCH
    pltpu.sync_copy(dstall.at[pl.ds(_mo8(w * EPW), EPW)], idxb)

    def fire(c, off):
        pltpu.async_copy(ya.at[idxb.at[pl.ds(_mo8(c * GCH), GCH)]],
                         rows.at[pl.ds(_mo8(off), GCH)], sem)

    fire(0, 0)

    def per_chunk(c, _):
        poff = (c & 1) * GCH
        qoff = GCH - poff
        pltpu.make_async_copy(ya.at[idxb.at[pl.ds(_mo8(c * GCH), GCH)]],
                              rows.at[pl.ds(_mo8(poff), GCH)], sem).wait()

        @pl.when(c + 1 < nch)
        def _():
            fire(c + 1, qoff)

        pltpu.sync_copy(rows.at[pl.ds(_mo8(poff), GCH)],
                        g.at[pl.ds(_mo8(w * EPW + c * GCH), GCH)])
        return 0

    lax.fori_loop(0, nch, per_chunk, 0)


def _gather_call(ya, dstall):
    return pl.kernel(
        _gather_body,
        compiler_params=_SC_PARAMS,
        interpret=INTERPRET,
        out_type=jax.ShapeDtypeStruct((E_ALL, H), F32),
        mesh=_mesh(),
        scratch_types=[
            pltpu.VMEM((EPW,), I32),
            pltpu.VMEM((2 * GCH, H), F32),
            pltpu.SemaphoreType.DMA,
        ],
    )(ya, dstall)


def _dot(a, b):
    return jnp.dot(a, b, preferred_element_type=F32)


def _full(shape):
    return pl.BlockSpec(shape, lambda i: (0, 0))


def _agg_max(a0, a1, a2):
    m = lambda a: jnp.where(jnp.isneginf(a), 0.0, a)
    return jnp.maximum(jnp.maximum(m(a0), m(a1)), m(a2))


def _embed_kernel(x, w0, b0, w1, b1, o):
    h = jnp.maximum(_dot(x[...], w0[...]) + b0[...], 0.0)
    o[...] = _dot(h, w1[...]) + b1[...]


def _embed_call(x, w0, b0, w1, b1, blk):
    n, d = x.shape
    return pl.pallas_call(
        _embed_kernel,
        grid=(n // blk,),
        in_specs=[pl.BlockSpec((blk, d), lambda i: (i, 0)),
                  _full(w0.shape), _full(b0.shape),
                  _full(w1.shape), _full(b1.shape)],
        out_specs=pl.BlockSpec((blk, H), lambda i: (i, 0)),
        out_shape=jax.ShapeDtypeStruct((n, H), F32),
        interpret=INTERPRET,
    )(x, w0, b0, w1, b1)


def _edge0_kernel(x, w0, b0, w1, b1, fw0, fb0, fw1, fb1, ea, msg):
    h = jnp.maximum(_dot(x[...], w0[...]) + b0[...], 0.0)
    e = _dot(h, w1[...]) + b1[...]
    ea[...] = e
    h2 = jnp.maximum(_dot(e, fw0[...]) + fb0[...], 0.0)
    msg[...] = _dot(h2, fw1[...]) + fb1[...]


def _edge0_call(x, w0, b0, w1, b1, fw0, fb0, fw1, fb1, blk):
    n, d = x.shape
    return pl.pallas_call(
        _edge0_kernel,
        grid=(n // blk,),
        in_specs=[pl.BlockSpec((blk, d), lambda i: (i, 0)),
                  _full(w0.shape), _full(b0.shape),
                  _full(w1.shape), _full(b1.shape),
                  _full(fw0.shape), _full(fb0.shape),
                  _full(fw1.shape), _full(fb1.shape)],
        out_specs=[pl.BlockSpec((blk, H), lambda i: (i, 0)),
                   pl.BlockSpec((blk, H), lambda i: (i, 0))],
        out_shape=[jax.ShapeDtypeStruct((n, H), F32),
                   jax.ShapeDtypeStruct((n, H), F32)],
        interpret=INTERPRET,
    )(x, w0, b0, w1, b1, fw0, fb0, fw1, fb1)


def _node_kernel(xa, a0, a1, a2, w0b, b0, xan, ya):
    x = xa[...] + _agg_max(a0[...], a1[...], a2[...])
    xan[...] = x
    ya[...] = _dot(x, w0b[...]) + b0[...]


def _node_call(xa, a0, a1, a2, w0b, b0, blk):
    n = xa.shape[0]
    bs = pl.BlockSpec((blk, H), lambda i: (i, 0))
    return pl.pallas_call(
        _node_kernel,
        grid=(n // blk,),
        in_specs=[bs, bs, bs, bs, _full(w0b.shape), _full(b0.shape)],
        out_specs=[bs, bs],
        out_shape=[jax.ShapeDtypeStruct((n, H), F32),
                   jax.ShapeDtypeStruct((n, H), F32)],
        interpret=INTERPRET,
    )(xa, a0, a1, a2, w0b, b0)


def _edge_kernel(ea, g, w0a, w1, b1, fw0, fb0, fw1, fb1, ean, msg):
    e = ea[...]
    t = jnp.maximum(_dot(e, w0a[...]) + g[...], 0.0)
    e = e + _dot(t, w1[...]) + b1[...]
    ean[...] = e
    h = jnp.maximum(_dot(e, fw0[...]) + fb0[...], 0.0)
    msg[...] = _dot(h, fw1[...]) + fb1[...]


def _edge_call(ea, g, w0a, w1, b1, fw0, fb0, fw1, fb1, blk):
    n = ea.shape[0]
    bs = pl.BlockSpec((blk, H), lambda i: (i, 0))
    return pl.pallas_call(
        _edge_kernel,
        grid=(n // blk,),
        in_specs=[bs, bs,
                  _full(w0a.shape), _full(w1.shape), _full(b1.shape),
                  _full(fw0.shape), _full(fb0.shape),
                  _full(fw1.shape), _full(fb1.shape)],
        out_specs=[bs, bs],
        out_shape=[jax.ShapeDtypeStruct((n, H), F32),
                   jax.ShapeDtypeStruct((n, H), F32)],
        interpret=INTERPRET,
    )(ea, g, w0a, w1, b1, fw0, fb0, fw1, fb1)


def _field_kernel(xa, a0, a1, a2, act, w0x, w0a, b0, w1p, b1p, o):
    x = xa[...] + _agg_max(a0[...], a1[...], a2[...])
    h = jnp.maximum(_dot(x, w0x[...]) + _dot(act[...], w0a[...]) + b0[...],
                    0.0)
    o[...] = _dot(h, w1p[...]) + b1p[...]


def _field_call(xa, a0, a1, a2, act, w0x, w0a, b0, w1p, b1p, blk):
    n = xa.shape[0]
    bs = pl.BlockSpec((blk, H), lambda i: (i, 0))
    return pl.pallas_call(
        _field_kernel,
        grid=(n // blk,),
        in_specs=[bs, bs, bs, bs,
                  pl.BlockSpec((blk, ACT), lambda i: (i, 0)),
                  _full(w0x.shape), _full(w0a.shape), _full(b0.shape),
                  _full(w1p.shape), _full(b1p.shape)],
        out_specs=bs,
        out_shape=jax.ShapeDtypeStruct((n, H), F32),
        interpret=INTERPRET,
    )(xa, a0, a1, a2, act, w0x, w0a, b0, w1p, b1p)


def kernel(x_obstacle, x_agent, x_goal, edge_index_oa, edge_index_aa,
           edge_index_ga, edge_attr_oa, edge_attr_aa, edge_attr_ga, action,
           params):
    p = params
    r1 = lambda b: b.reshape(1, -1)

    dst_all = jnp.concatenate(
        [edge_index_oa[1], edge_index_aa[1], edge_index_ga[1]])
    ecat = jnp.concatenate([edge_attr_oa, edge_attr_aa, edge_attr_ga], axis=0)

    xa = _embed_call(x_agent, p["embed_W0"], r1(p["embed_b0"]),
                     p["embed_W1"], r1(p["embed_b1"]), BLK_N)

    ea, msg = _edge0_call(ecat, p["eembed_W0"], r1(p["eembed_b0"]),
                          p["eembed_W1"], r1(p["eembed_b1"]),
                          p["fx0_W0"], r1(p["fx0_b0"]),
                          p["fx0_W1"], r1(p["fx0_b1"]), BLK_E)

    slist, mcnt = _match_call(dst_all)

    for l in range(3):
        aggs = _seg_call(msg, slist, mcnt).reshape(N_TYPES, NPAD, H)
        a0, a1, a2 = (aggs[t][:N_NODE] for t in range(N_TYPES))
        if l == 2:
            w0 = p["field_W0"]
            w1p = jnp.pad(p["field_W1"], ((0, 0), (0, H - 1)))
            b1p = jnp.broadcast_to(p["field_b1"].reshape(1, 1), (1, H))
            out = _field_call(xa, a0, a1, a2, action, w0[:H], w0[H:],
                              r1(p["field_b0"]), w1p, b1p, BLK_N)
            return out[:, 0]
        em = "em%d_" % l
        w0 = p[em + "W0"]
        xa, ya = _node_call(xa, a0, a1, a2, w0[H:], r1(p[em + "b0"]), BLK_N)
        g = _gather_call(ya, dst_all)
        fx = "fx%d_" % (l + 1)
        ea, msg = _edge_call(ea, g, w0[:H], p[em + "W1"], r1(p[em + "b1"]),
                             p[fx + "W0"], r1(p[fx + "b0"]),
                             p[fx + "W1"], r1(p[fx + "b1"]), BLK_E)


# speedup vs baseline: 20.4041x; 1.0016x over previous
import functools

import jax
import jax.numpy as jnp
from jax import lax
from jax.experimental import pallas as pl
from jax.experimental.pallas import tpu as pltpu
from jax.experimental.pallas import tpu_sc as plsc

F32 = jnp.float32
BF16 = jnp.bfloat16
I32 = jnp.int32

N_NODE = 10000
E_T = 160000
N_TYPES = 3
E_ALL = N_TYPES * E_T
H = 128
ACT = 8

NC = 2
NS = 16
NW = NC * NS

RNG = 320
NPAD = NW * RNG
ACCROWS = RNG + 1
SCRAP = RNG
SCAN_CHUNK = 3200
N_SCAN = E_T // SCAN_CHUNK
ROUND = 16384
MB_CAP = ROUND + SCAN_CHUNK + 160
HB = 336
CAP = 160384
RMW_CHUNK = 128
EPW = E_ALL // NW
GCH = 120
BLK_E = 8000
BLK_N = 10000
INTERPRET = False


def _wid():
    return lax.axis_index("s") * NC + lax.axis_index("c")


def _mesh():
    return plsc.VectorSubcoreMesh(core_axis_name="c", subcore_axis_name="s",
                                  num_cores=NC, num_subcores=NS)


_SC_PARAMS = pltpu.CompilerParams(needs_layout_passes=False)


def _mo8(x):
    return pl.multiple_of(x, 8)


def _match_body(dstall, slist, mcnt, dstbuf, mbuf, sortbuf, histo, posb,
                cbuf):
    w = _wid()
    iota = lax.iota(I32, 16)
    node_base = w * RNG
    zeros = jnp.zeros((16,), I32)
    sentinel = zeros + (SCRAP << 19)

    def shuf(x, idx):
        dn = lax.GatherDimensionNumbers(offset_dims=(),
                                        collapsed_slice_dims=(0,),
                                        start_index_map=(0,))
        return lax.gather(x, idx[:, None], dn, (1,),
                          mode=lax.GatherScatterMode.PROMISE_IN_BOUNDS)

    def run_info(dl):
        prev = shuf(dl, jnp.maximum(iota - 1, 0))
        nxt = shuf(dl, jnp.minimum(iota + 1, 15))
        is_start = (dl != prev) | (iota == 0)
        is_end = (dl != nxt) | (iota == 15)
        run_start = plsc.cummax(jnp.where(is_start, iota, zeros))
        rank = iota - run_start
        return rank, is_end

    def per_type(t, _):
        base0 = (t * NW + w) * CAP

        def sort_flush(nv, G):
            for k in range(HB // 16):
                histo[pl.ds(k * 16, 16)] = zeros

            def hist(g, _):
                v = mbuf[pl.ds(g * 16, 16)]
                dl = lax.sort(v) >> 19
                rank, is_end = run_info(dl)
                plsc.addupdate_scatter(histo, [dl], rank + 1, mask=is_end)
                return 0

            lax.fori_loop(0, nv, hist, 0)
            carry = zeros
            for k in range(HB // 16):
                v = histo[pl.ds(k * 16, 16)]
                inc = plsc.cumsum(v)
                posb[pl.ds(k * 16, 16)] = carry + inc - v
                carry = carry + (zeros + inc[15])

            def perm(g, _):
                sv = lax.sort(mbuf[pl.ds(g * 16, 16)])
                dl = sv >> 19
                rank, is_end = run_info(dl)
                base = plsc.load_gather(posb, [dl])
                plsc.store_scatter(sortbuf, [base + rank], sv)
                plsc.addupdate_scatter(posb, [dl], rank + 1, mask=is_end)
                return 0

            lax.fori_loop(0, nv, perm, 0)

            def flush(k, _):
                pltpu.sync_copy(
                    sortbuf.at[pl.ds(k * 128, 128)],
                    slist.at[pl.ds(_mo8(base0 + G + k * 128), 128)])
                return 0

            lax.fori_loop(0, nv // 8, flush, 0)

        def per_chunk(c, carry):
            G, cntv = carry
            pltpu.sync_copy(
                dstall.at[pl.ds(_mo8(t * E_T + c * SCAN_CHUNK), SCAN_CHUNK)],
                dstbuf)

            def scan(i, cntv):
                vecs = []
                for u in range(4):
                    idx = i * 64 + u * 16 + iota
                    v = plsc.load_gather(dstbuf, [idx])
                    dl = v - node_base
                    mask = (dl >= 0) & (dl < RNG)
                    eidv = t * E_T + c * SCAN_CHUNK + idx
                    packed = (dl << 19) + eidv
                    cs = plsc.cumsum(mask.astype(I32))
                    pc = plsc.all_reduce_population_count(mask)
                    vecs.append((packed, mask, cs, pc))
                for packed, mask, cs, pc in vecs:
                    plsc.store_scatter(mbuf, [cntv + cs - 1], packed,
                                       mask=mask)
                    cntv = cntv + pc
                return cntv

            cntv = lax.fori_loop(0, SCAN_CHUNK // 64, scan, cntv)
            cnt = jnp.max(cntv)

            def do_round(args):
                G, cntv = args
                sort_flush(ROUND // 16, G)
                rem = cnt - ROUND

                def shift(g, _):
                    mv = plsc.load_gather(mbuf, [ROUND + g * 16 + iota])
                    plsc.store_scatter(mbuf, [g * 16 + iota], mv)
                    return 0

                lax.fori_loop(0, (rem + 15) // 16, shift, 0)
                return G + ROUND, cntv - ROUND

            return lax.cond(cnt >= ROUND, do_round, lambda a: a, (G, cntv))

        G, cntv = lax.fori_loop(0, N_SCAN, per_chunk, (0, zeros))
        cnt = jnp.max(cntv)
        for k in range(128 // 16):
            plsc.store_scatter(mbuf, [cnt + k * 16 + iota], sentinel)
        cnt_pad = ((cnt + 127) // 128) * 128
        sort_flush(cnt_pad // 16, G)
        G = G + cnt_pad
        cbuf[...] = zeros + G
        pltpu.sync_copy(cbuf, mcnt.at[pl.ds(_mo8((t * NW + w) * 16), 16)])
        return 0

    lax.fori_loop(0, N_TYPES, per_type, 0)


def _match_call(dstall):
    return pl.kernel(
        _match_body,
        compiler_params=_SC_PARAMS,
        interpret=INTERPRET,
        out_type=(jax.ShapeDtypeStruct((N_TYPES * NW * CAP,), I32),
                  jax.ShapeDtypeStruct((N_TYPES * NW * 16,), I32)),
        mesh=_mesh(),
        scratch_types=[
            pltpu.VMEM((SCAN_CHUNK,), I32),
            pltpu.VMEM((MB_CAP,), I32),
            pltpu.VMEM((ROUND,), I32),
            pltpu.VMEM((HB,), I32),
            pltpu.VMEM((HB,), I32),
            pltpu.VMEM((16,), I32),
        ],
    )(dstall)


def _seg_body(msg, slist, mcnt, aggs, acc, pbulk, pbuf, idxbuf, rows,
              cntbuf, sem):
    w = _wid()
    neg = jnp.full((16,), -jnp.inf, F32)
    BULK = 32768

    def _off(cur, f):
        return cur * H + f * 16

    def per_type(t, _):
        def init_acc(i, _):
            acc[pl.ds(i * 16, 16)] = neg
            return 0

        lax.fori_loop(0, (ACCROWS * H) // 16, init_acc, 0)
        pltpu.sync_copy(mcnt.at[pl.ds(_mo8((t * NW + w) * 16), 16)], cntbuf)
        cnt = jnp.max(cntbuf[...])
        base0 = (t * NW + w) * CAP
        nch = cnt // RMW_CHUNK
        init = (jnp.int32(SCRAP),) + (neg,) * 8

        def mk_loop(src, src_off, pre_stage):
            def stage(c, off):
                for i in range(RMW_CHUNK // 16):
                    v = src[pl.ds(src_off(c) + i * 16, 16)]
                    idxbuf[pl.ds(off + i * 16, 16)] = v & 0x7FFFF
                pltpu.async_copy(
                    msg.at[idxbuf.at[pl.ds(off, RMW_CHUNK)]],
                    rows.at[pl.ds(_mo8(off), RMW_CHUNK)], sem)

            @pl.when(nch > 0)
            def _():
                pre_stage(0)
                stage(0, 0)

            def per_chunk(c, carry):
                poff = (c & 1) * RMW_CHUNK
                qoff = RMW_CHUNK - poff
                pltpu.make_async_copy(
                    msg.at[idxbuf.at[pl.ds(poff, RMW_CHUNK)]],
                    rows.at[pl.ds(_mo8(poff), RMW_CHUNK)], sem).wait()

                @pl.when(c + 1 < nch)
                def _():
                    pre_stage(c + 1)
                    stage(c + 1, qoff)

                def group(g, carry):
                    cur = carry[0]
                    regs = list(carry[1:])
                    v = src[pl.ds(src_off(c) + g * 16, 16)]
                    for k in range(16):
                        dl = v[k] >> 19
                        fl = dl != cur
                        cur_old = cur
                        regs_old = tuple(regs)

                        @pl.when(fl)
                        def _():
                            for f in range(8):
                                a = acc[pl.ds(_off(cur_old, f), 16)]
                                acc[pl.ds(_off(cur_old, f), 16)] = (
                                    jnp.maximum(a, regs_old[f]))

                        j = poff + g * 16 + k
                        for f in range(8):
                            r = rows[j, pl.ds(f * 16, 16)]
                            regs[f] = jnp.where(fl, r,
                                                jnp.maximum(regs[f], r))
                        cur = jnp.where(fl, dl, cur)
                    return (cur, *regs)

                return lax.fori_loop(0, RMW_CHUNK // 16, group, carry)

            return lax.fori_loop(0, nch, per_chunk, init)

        def fast(_):
            def load2k(k, _):
                pltpu.sync_copy(
                    slist.at[pl.ds(_mo8(base0 + k * 2048), 2048)],
                    pbulk.at[pl.ds(k * 2048, 2048)])
                return 0

            lax.fori_loop(0, cnt // 2048, load2k, 0)

            def load128(k, _):
                pltpu.sync_copy(
                    slist.at[pl.ds(_mo8(base0 + k * 128), 128)],
                    pbulk.at[pl.ds(k * 128, 128)])
                return 0

            lax.fori_loop((cnt // 2048) * 16, nch, load128, 0)
            return mk_loop(pbulk, lambda c: c * RMW_CHUNK, lambda c: None)

        def slow(_):
            def pre_stage(c):
                pltpu.sync_copy(
                    slist.at[pl.ds(_mo8(base0 + c * RMW_CHUNK), RMW_CHUNK)],
                    pbuf.at[pl.ds((c & 1) * RMW_CHUNK, RMW_CHUNK)])

            return mk_loop(pbuf, lambda c: (c & 1) * RMW_CHUNK, pre_stage)

        fin = lax.cond(cnt <= BULK, fast, slow, 0)
        cur = fin[0]
        for f in range(8):
            a = acc[pl.ds(_off(cur, f), 16)]
            acc[pl.ds(_off(cur, f), 16)] = jnp.maximum(a, fin[1 + f])
        pltpu.sync_copy(
            acc.at[pl.ds(0, RNG * H)],
            aggs.at[pl.ds(_mo8((t * NPAD + w * RNG) * H), RNG * H)])
        return 0

    lax.fori_loop(0, N_TYPES, per_type, 0)


def _seg_call(msg, slist, mcnt):
    return pl.kernel(
        _seg_body,
        compiler_params=_SC_PARAMS,
        interpret=INTERPRET,
        out_type=jax.ShapeDtypeStruct((N_TYPES * NPAD * H,), F32),
        mesh=_mesh(),
        scratch_types=[
            pltpu.VMEM((ACCROWS * H,), F32),
            pltpu.VMEM((32768,), I32),
            pltpu.VMEM((2 * RMW_CHUNK,), I32),
            pltpu.VMEM((2 * RMW_CHUNK,), I32),
            pltpu.VMEM((2 * RMW_CHUNK, H), F32),
            pltpu.VMEM((16,), I32),
            pltpu.SemaphoreType.DMA,
        ],
    )(msg, slist, mcnt)


def _gather_body(ya, dstall, g, idxb, rows, sem):
    w = _wid()
    nch = EPW // GCH
    pltpu.sync_copy(dstall.at[pl.ds(_mo8(w * EPW), EPW)], idxb)

    def fire(c, off):
        pltpu.async_copy(ya.at[idxb.at[pl.ds(_mo8(c * GCH), GCH)]],
                         rows.at[pl.ds(_mo8(off), GCH)], sem)

    fire(0, 0)

    def per_chunk(c, _):
        poff = (c & 1) * GCH
        qoff = GCH - poff
        pltpu.make_async_copy(ya.at[idxb.at[pl.ds(_mo8(c * GCH), GCH)]],
                              rows.at[pl.ds(_mo8(poff), GCH)], sem).wait()

        @pl.when(c + 1 < nch)
        def _():
            fire(c + 1, qoff)

        pltpu.sync_copy(rows.at[pl.ds(_mo8(poff), GCH)],
                        g.at[pl.ds(_mo8(w * EPW + c * GCH), GCH)])
        return 0

    lax.fori_loop(0, nch, per_chunk, 0)


def _gather_call(ya, dstall):
    return pl.kernel(
        _gather_body,
        compiler_params=_SC_PARAMS,
        interpret=INTERPRET,
        out_type=jax.ShapeDtypeStruct((E_ALL, H), F32),
        mesh=_mesh(),
        scratch_types=[
            pltpu.VMEM((EPW,), I32),
            pltpu.VMEM((2 * GCH, H), F32),
            pltpu.SemaphoreType.DMA,
        ],
    )(ya, dstall)


def _dot(a, b):
    return jnp.dot(a, b, preferred_element_type=F32)


def _full(shape):
    return pl.BlockSpec(shape, lambda i: (0, 0))


def _agg_max(a0, a1, a2):
    m = lambda a: jnp.where(jnp.isneginf(a), 0.0, a)
    return jnp.maximum(jnp.maximum(m(a0), m(a1)), m(a2))


def _embed_kernel(x, w0, b0, w1, b1, o):
    h = jnp.maximum(_dot(x[...], w0[...]) + b0[...], 0.0)
    o[...] = _dot(h, w1[...]) + b1[...]


def _embed_call(x, w0, b0, w1, b1, blk):
    n, d = x.shape
    return pl.pallas_call(
        _embed_kernel,
        grid=(n // blk,),
        in_specs=[pl.BlockSpec((blk, d), lambda i: (i, 0)),
                  _full(w0.shape), _full(b0.shape),
                  _full(w1.shape), _full(b1.shape)],
        out_specs=pl.BlockSpec((blk, H), lambda i: (i, 0)),
        out_shape=jax.ShapeDtypeStruct((n, H), F32),
        interpret=INTERPRET,
    )(x, w0, b0, w1, b1)


def _edge0_kernel(x, w0, b0, w1, b1, fw0, fb0, fw1, fb1, ea, msg):
    h = jnp.maximum(_dot(x[...], w0[...]) + b0[...], 0.0)
    e = _dot(h, w1[...]) + b1[...]
    ea[...] = e
    h2 = jnp.maximum(_dot(e, fw0[...]) + fb0[...], 0.0)
    msg[...] = _dot(h2, fw1[...]) + fb1[...]


def _edge0_call(x, w0, b0, w1, b1, fw0, fb0, fw1, fb1, blk):
    n, d = x.shape
    return pl.pallas_call(
        _edge0_kernel,
        grid=(n // blk,),
        in_specs=[pl.BlockSpec((blk, d), lambda i: (i, 0)),
                  _full(w0.shape), _full(b0.shape),
                  _full(w1.shape), _full(b1.shape),
                  _full(fw0.shape), _full(fb0.shape),
                  _full(fw1.shape), _full(fb1.shape)],
        out_specs=[pl.BlockSpec((blk, H), lambda i: (i, 0)),
                   pl.BlockSpec((blk, H), lambda i: (i, 0))],
        out_shape=[jax.ShapeDtypeStruct((n, H), F32),
                   jax.ShapeDtypeStruct((n, H), F32)],
        interpret=INTERPRET,
    )(x, w0, b0, w1, b1, fw0, fb0, fw1, fb1)


def _node_kernel(xa, a0, a1, a2, w0b, b0, xan, ya):
    x = xa[...] + _agg_max(a0[...], a1[...], a2[...])
    xan[...] = x
    ya[...] = _dot(x, w0b[...]) + b0[...]


def _node_call(xa, a0, a1, a2, w0b, b0, blk):
    n = xa.shape[0]
    bs = pl.BlockSpec((blk, H), lambda i: (i, 0))
    return pl.pallas_call(
        _node_kernel,
        grid=(n // blk,),
        in_specs=[bs, bs, bs, bs, _full(w0b.shape), _full(b0.shape)],
        out_specs=[bs, bs],
        out_shape=[jax.ShapeDtypeStruct((n, H), F32),
                   jax.ShapeDtypeStruct((n, H), F32)],
        interpret=INTERPRET,
    )(xa, a0, a1, a2, w0b, b0)


def _edge_kernel(ea, g, w0a, w1, b1, fw0, fb0, fw1, fb1, ean, msg):
    e = ea[...]
    t = jnp.maximum(_dot(e, w0a[...]) + g[...], 0.0)
    e = e + _dot(t, w1[...]) + b1[...]
    ean[...] = e
    h = jnp.maximum(_dot(e, fw0[...]) + fb0[...], 0.0)
    msg[...] = _dot(h, fw1[...]) + fb1[...]


def _edge_call(ea, g, w0a, w1, b1, fw0, fb0, fw1, fb1, blk):
    n = ea.shape[0]
    bs = pl.BlockSpec((blk, H), lambda i: (i, 0))
    return pl.pallas_call(
        _edge_kernel,
        grid=(n // blk,),
        in_specs=[bs, bs,
                  _full(w0a.shape), _full(w1.shape), _full(b1.shape),
                  _full(fw0.shape), _full(fb0.shape),
                  _full(fw1.shape), _full(fb1.shape)],
        out_specs=[bs, bs],
        out_shape=[jax.ShapeDtypeStruct((n, H), F32),
                   jax.ShapeDtypeStruct((n, H), F32)],
        interpret=INTERPRET,
    )(ea, g, w0a, w1, b1, fw0, fb0, fw1, fb1)


def _field_kernel(xa, a0, a1, a2, act, w0x, w0a, b0, w1p, b1p, o):
    x = xa[...] + _agg_max(a0[...], a1[...], a2[...])
    h = jnp.maximum(_dot(x, w0x[...]) + _dot(act[...], w0a[...]) + b0[...],
                    0.0)
    o[...] = _dot(h, w1p[...]) + b1p[...]


def _field_call(xa, a0, a1, a2, act, w0x, w0a, b0, w1p, b1p, blk):
    n = xa.shape[0]
    bs = pl.BlockSpec((blk, H), lambda i: (i, 0))
    return pl.pallas_call(
        _field_kernel,
        grid=(n // blk,),
        in_specs=[bs, bs, bs, bs,
                  pl.BlockSpec((blk, ACT), lambda i: (i, 0)),
                  _full(w0x.shape), _full(w0a.shape), _full(b0.shape),
                  _full(w1p.shape), _full(b1p.shape)],
        out_specs=bs,
        out_shape=jax.ShapeDtypeStruct((n, H), F32),
        interpret=INTERPRET,
    )(xa, a0, a1, a2, act, w0x, w0a, b0, w1p, b1p)


def kernel(x_obstacle, x_agent, x_goal, edge_index_oa, edge_index_aa,
           edge_index_ga, edge_attr_oa, edge_attr_aa, edge_attr_ga, action,
           params):
    p = params
    r1 = lambda b: b.reshape(1, -1)

    dst_all = jnp.concatenate(
        [edge_index_oa[1], edge_index_aa[1], edge_index_ga[1]])
    ecat = jnp.concatenate([edge_attr_oa, edge_attr_aa, edge_attr_ga], axis=0)

    xa = _embed_call(x_agent, p["embed_W0"], r1(p["embed_b0"]),
                     p["embed_W1"], r1(p["embed_b1"]), BLK_N)

    ea, msg = _edge0_call(ecat, p["eembed_W0"], r1(p["eembed_b0"]),
                          p["eembed_W1"], r1(p["eembed_b1"]),
                          p["fx0_W0"], r1(p["fx0_b0"]),
                          p["fx0_W1"], r1(p["fx0_b1"]), BLK_E)

    slist, mcnt = _match_call(dst_all)

    for l in range(3):
        aggs = _seg_call(msg, slist, mcnt).reshape(N_TYPES, NPAD, H)
        a0, a1, a2 = (aggs[t][:N_NODE] for t in range(N_TYPES))
        if l == 2:
            w0 = p["field_W0"]
            w1p = jnp.pad(p["field_W1"], ((0, 0), (0, H - 1)))
            b1p = jnp.broadcast_to(p["field_b1"].reshape(1, 1), (1, H))
            out = _field_call(xa, a0, a1, a2, action, w0[:H], w0[H:],
                              r1(p["field_b0"]), w1p, b1p, BLK_N)
            return out[:, 0]
        em = "em%d_" % l
        w0 = p[em + "W0"]
        xa, ya = _node_call(xa, a0, a1, a2, w0[H:], r1(p[em + "b0"]), BLK_N)
        g = _gather_call(ya, dst_all)
        fx = "fx%d_" % (l + 1)
        ea, msg = _edge_call(ea, g, w0[:H], p[em + "W1"], r1(p[em + "b1"]),
                             p[fx + "W0"], r1(p[fx + "b0"]),
                             p[fx + "W1"], r1(p[fx + "b1"]), BLK_E)
